# baseline (speedup 1.0000x reference)
import jax
import jax.numpy as jnp
from jax.experimental import pallas as pl
from jax.experimental.pallas import tpu as pltpu

N = 16384
C = 1000
K = int(0.7 * N)
IGNORE = 0

B = 1024
NB = N // B
CH = 8
NCH = C // CH


def _body(xt_ref, tgt_ref, out_ref, loss_ref):
    i = pl.program_id(0)
    tgt = tgt_ref[...]

    acc = xt_ref[pl.ds(0, CH), :]
    for j in range(1, NCH):
        acc = jnp.maximum(acc, xt_ref[pl.ds(CH * j, CH), :])
    m = jnp.max(acc, axis=0, keepdims=True)

    rows8 = jax.lax.broadcasted_iota(jnp.int32, (CH, B), 0)
    oh8 = (rows8 == tgt % CH).astype(jnp.float32)
    tgtc = tgt // CH
    s8 = jnp.zeros((CH, B), jnp.float32)
    sel8 = jnp.zeros((CH, B), jnp.float32)
    for j in range(NCH):
        ch = xt_ref[pl.ds(CH * j, CH), :]
        s8 = s8 + jnp.exp(ch - m)
        sel8 = sel8 + ch * (tgtc == j).astype(jnp.float32)
    s = jnp.sum(s8, axis=0, keepdims=True)
    picked = jnp.sum(sel8 * oh8, axis=0, keepdims=True)
    lse = m + jnp.log(s)
    loss_ref[pl.ds(i, 1), :] = jnp.where(tgt == IGNORE, 0.0, lse - picked)

    @pl.when(i == NB - 1)
    def _select():
        lx = loss_ref[...]
        bits = jax.lax.bitcast_convert_type(lx, jnp.int32)

        def srch(_, carry):
            lo, hi = carry
            mid = lo + (hi - lo) // 2
            cnt = jnp.sum(jnp.where(bits >= mid, 1, 0))
            return (jnp.where(cnt >= K, mid, lo),
                    jnp.where(cnt >= K, hi, mid))

        t, _ = jax.lax.fori_loop(
            0, 31, srch, (jnp.int32(0), jnp.int32(0x7F800001)))
        gt = bits > t
        cnt_gt = jnp.sum(jnp.where(gt, 1.0, 0.0))
        sum_gt = jnp.sum(jnp.where(gt, lx, 0.0))
        tv = jnp.max(jax.lax.bitcast_convert_type(
            jnp.full((8, 128), t, jnp.int32), jnp.float32))
        out_ref[0, 0] = (sum_gt + (jnp.float32(K) - cnt_gt) * tv) * (1.0 / K)


@jax.jit
def kernel(input, target):
    xt = input.T
    tgt2d = target.astype(jnp.int32)[None, :]

    out = pl.pallas_call(
        _body,
        grid=(NB,),
        in_specs=[
            pl.BlockSpec((C, B), lambda i: (0, i)),
            pl.BlockSpec((1, B), lambda i: (0, i)),
        ],
        out_specs=pl.BlockSpec(memory_space=pltpu.SMEM),
        out_shape=jax.ShapeDtypeStruct((1, 1), jnp.float32),
        scratch_shapes=[pltpu.VMEM((NB, B), jnp.float32)],
    )(xt, tgt2d)
    return out[0, 0]


# speedup vs baseline: 1.1061x; 1.1061x over previous
import jax
import jax.numpy as jnp
from jax.experimental import pallas as pl
from jax.experimental.pallas import tpu as pltpu

N = 16384
C = 1000
K = int(0.7 * N)
IGNORE = 0

B = 2048
NB = N // B
CH = 8
NCH = C // CH


def _body(xt_ref, tgt_ref, out_ref, loss_ref):
    i = pl.program_id(0)
    tgt = tgt_ref[...]

    rows8 = jax.lax.broadcasted_iota(jnp.int32, (CH, B), 0)
    oh8 = (rows8 == tgt % CH).astype(jnp.float32)
    tgtc = tgt // CH

    acc = xt_ref[pl.ds(0, CH), :]
    sel8 = acc * (tgtc == 0).astype(jnp.float32)
    for j in range(1, NCH):
        ch = xt_ref[pl.ds(CH * j, CH), :]
        acc = jnp.maximum(acc, ch)
        sel8 = sel8 + ch * (tgtc == j).astype(jnp.float32)
    m = jnp.max(acc, axis=0, keepdims=True)

    s8 = jnp.zeros((CH, B), jnp.float32)
    for j in range(NCH):
        s8 = s8 + jnp.exp(xt_ref[pl.ds(CH * j, CH), :] - m)
    s = jnp.sum(s8, axis=0, keepdims=True)
    picked = jnp.sum(sel8 * oh8, axis=0, keepdims=True)
    lse = m + jnp.log(s)
    loss_ref[pl.ds(i, 1), :] = jnp.where(tgt == IGNORE, 0.0, lse - picked)

    @pl.when(i == NB - 1)
    def _select():
        lx = loss_ref[...]
        bits = jax.lax.bitcast_convert_type(lx, jnp.int32)

        def srch(_, carry):
            lo, hi = carry
            mid = lo + (hi - lo) // 2
            cnt = jnp.sum(jnp.where(bits >= mid, 1, 0))
            return (jnp.where(cnt >= K, mid, lo),
                    jnp.where(cnt >= K, hi, mid))

        t, _ = jax.lax.fori_loop(
            0, 31, srch, (jnp.int32(0), jnp.int32(0x7F800001)))
        gt = bits > t
        cnt_gt = jnp.sum(jnp.where(gt, 1.0, 0.0))
        sum_gt = jnp.sum(jnp.where(gt, lx, 0.0))
        tv = jnp.max(jax.lax.bitcast_convert_type(
            jnp.full((8, 128), t, jnp.int32), jnp.float32))
        out_ref[0, 0] = (sum_gt + (jnp.float32(K) - cnt_gt) * tv) * (1.0 / K)


@jax.jit
def kernel(input, target):
    xt = input.T
    tgt2d = target.astype(jnp.int32)[None, :]

    out = pl.pallas_call(
        _body,
        grid=(NB,),
        in_specs=[
            pl.BlockSpec((C, B), lambda i: (0, i)),
            pl.BlockSpec((1, B), lambda i: (0, i)),
        ],
        out_specs=pl.BlockSpec(memory_space=pltpu.SMEM),
        out_shape=jax.ShapeDtypeStruct((1, 1), jnp.float32),
        scratch_shapes=[pltpu.VMEM((NB, B), jnp.float32)],
    )(xt, tgt2d)
    return out[0, 0]


# speedup vs baseline: 1.1663x; 1.0545x over previous
import jax
import jax.numpy as jnp
from jax.experimental import pallas as pl
from jax.experimental.pallas import tpu as pltpu

N = 16384
C = 1000
K = int(0.7 * N)
IGNORE = 0

B = 2048
NB = N // B
CH = 8
NCH = C // CH


def _body(xt_ref, tgt_ref, out_ref, loss_ref):
    i = pl.program_id(0)
    tgt = tgt_ref[...]

    rows8 = jax.lax.broadcasted_iota(jnp.int32, (CH, B), 0)
    oh8 = (rows8 == tgt % CH).astype(jnp.float32)
    tgtc = tgt // CH

    acc = xt_ref[pl.ds(0, CH), :]
    sel8 = acc * (tgtc == 0).astype(jnp.float32)
    for j in range(1, NCH):
        ch = xt_ref[pl.ds(CH * j, CH), :]
        acc = jnp.maximum(acc, ch)
        sel8 = sel8 + ch * (tgtc == j).astype(jnp.float32)
    m = jnp.max(acc, axis=0, keepdims=True)

    s8 = jnp.zeros((CH, B), jnp.float32)
    for j in range(NCH):
        s8 = s8 + jnp.exp(xt_ref[pl.ds(CH * j, CH), :] - m)
    s = jnp.sum(s8, axis=0, keepdims=True)
    picked = jnp.sum(sel8 * oh8, axis=0, keepdims=True)
    lse = m + jnp.log(s)
    loss_ref[pl.ds(i, 1), :] = jnp.where(tgt == IGNORE, 0.0, lse - picked)

    @pl.when(i == NB - 1)
    def _select():
        lx = loss_ref[...]
        bits = jax.lax.bitcast_convert_type(lx, jnp.int32)

        def srch(_, carry):
            lo, hi = carry
            q = (hi - lo) // 4
            m1 = lo + q
            m2 = lo + 2 * q
            m3 = lo + 3 * q
            c1 = jnp.sum(jnp.where(bits >= m1, 1, 0))
            c2 = jnp.sum(jnp.where(bits >= m2, 1, 0))
            c3 = jnp.sum(jnp.where(bits >= m3, 1, 0))
            lo2 = jnp.where(c3 >= K, m3,
                            jnp.where(c2 >= K, m2,
                                      jnp.where(c1 >= K, m1, lo)))
            hi2 = jnp.where(c1 < K, m1,
                            jnp.where(c2 < K, m2,
                                      jnp.where(c3 < K, m3, hi)))
            return lo2, hi2

        def srch2(_, carry):
            lo, hi = carry
            mid = lo + (hi - lo) // 2
            cnt = jnp.sum(jnp.where(bits >= mid, 1, 0))
            return (jnp.where(cnt >= K, mid, lo),
                    jnp.where(cnt >= K, hi, mid))

        lohi = jax.lax.fori_loop(
            0, 16, srch, (jnp.int32(0), jnp.int32(0x7F800001)))
        t, _ = jax.lax.fori_loop(0, 2, srch2, lohi)
        gt = bits > t
        cnt_gt = jnp.sum(jnp.where(gt, 1.0, 0.0))
        sum_gt = jnp.sum(jnp.where(gt, lx, 0.0))
        tv = jnp.max(jax.lax.bitcast_convert_type(
            jnp.full((8, 128), t, jnp.int32), jnp.float32))
        out_ref[0, 0] = (sum_gt + (jnp.float32(K) - cnt_gt) * tv) * (1.0 / K)


@jax.jit
def kernel(input, target):
    xt = input.T
    tgt2d = target.astype(jnp.int32)[None, :]

    out = pl.pallas_call(
        _body,
        grid=(NB,),
        in_specs=[
            pl.BlockSpec((C, B), lambda i: (0, i)),
            pl.BlockSpec((1, B), lambda i: (0, i)),
        ],
        out_specs=pl.BlockSpec(memory_space=pltpu.SMEM),
        out_shape=jax.ShapeDtypeStruct((1, 1), jnp.float32),
        scratch_shapes=[pltpu.VMEM((NB, B), jnp.float32)],
    )(xt, tgt2d)
    return out[0, 0]


# speedup vs baseline: 1.1812x; 1.0128x over previous
import jax
import jax.numpy as jnp
from jax.experimental import pallas as pl
from jax.experimental.pallas import tpu as pltpu

N = 16384
C = 1000
K = int(0.7 * N)
IGNORE = 0

B = 2048
NB = N // B
CH = 8
NCH = C // CH


def _body(xt_ref, tgt_ref, out_ref, loss_ref):
    i = pl.program_id(0)
    tgt = tgt_ref[...]

    rows8 = jax.lax.broadcasted_iota(jnp.int32, (CH, B), 0)
    oh8 = (rows8 == tgt % CH).astype(jnp.float32)
    tgtc = tgt // CH

    acc = xt_ref[pl.ds(0, CH), :]
    sel8 = acc * (tgtc == 0).astype(jnp.float32)
    for j in range(1, NCH):
        ch = xt_ref[pl.ds(CH * j, CH), :]
        acc = jnp.maximum(acc, ch)
        sel8 = sel8 + ch * (tgtc == j).astype(jnp.float32)
    m = jnp.max(acc, axis=0, keepdims=True)

    s8 = jnp.zeros((CH, B), jnp.float32)
    for j in range(NCH):
        s8 = s8 + jnp.exp(xt_ref[pl.ds(CH * j, CH), :] - m)
    s = jnp.sum(s8, axis=0, keepdims=True)
    picked = jnp.sum(sel8 * oh8, axis=0, keepdims=True)
    lse = m + jnp.log(s)
    loss_ref[pl.ds(i, 1), :] = jnp.where(tgt == IGNORE, 0.0, lse - picked)

    @pl.when(i == NB - 1)
    def _select():
        lx = loss_ref[...]
        bits = jax.lax.bitcast_convert_type(lx, jnp.int32)

        def srch(_, carry):
            lo, hi = carry
            q = (hi - lo) // 8
            ms = [lo + n * q for n in range(1, 8)]
            cs = [jnp.sum(jnp.where(bits >= mn, 1, 0)) for mn in ms]
            lo2 = lo
            hi2 = hi
            for mn, cn in zip(ms, cs):
                lo2 = jnp.where(cn >= K, mn, lo2)
                hi2 = jnp.where(jnp.logical_and(cn < K, hi2 == hi), mn, hi2)
            return lo2, hi2

        def srch2(_, carry):
            lo, hi = carry
            mid = lo + (hi - lo) // 2
            cnt = jnp.sum(jnp.where(bits >= mid, 1, 0))
            return (jnp.where(cnt >= K, mid, lo),
                    jnp.where(cnt >= K, hi, mid))

        lohi = jax.lax.fori_loop(
            0, 10, srch, (jnp.int32(0), jnp.int32(0x7F800001)))
        t, _ = jax.lax.fori_loop(0, 4, srch2, lohi)
        gt = bits > t
        cnt_gt = jnp.sum(jnp.where(gt, 1.0, 0.0))
        sum_gt = jnp.sum(jnp.where(gt, lx, 0.0))
        tv = jnp.max(jax.lax.bitcast_convert_type(
            jnp.full((8, 128), t, jnp.int32), jnp.float32))
        out_ref[0, 0] = (sum_gt + (jnp.float32(K) - cnt_gt) * tv) * (1.0 / K)


@jax.jit
def kernel(input, target):
    xt = input.T
    tgt2d = target.astype(jnp.int32)[None, :]

    out = pl.pallas_call(
        _body,
        grid=(NB,),
        in_specs=[
            pl.BlockSpec((C, B), lambda i: (0, i)),
            pl.BlockSpec((1, B), lambda i: (0, i)),
        ],
        out_specs=pl.BlockSpec(memory_space=pltpu.SMEM),
        out_shape=jax.ShapeDtypeStruct((1, 1), jnp.float32),
        scratch_shapes=[pltpu.VMEM((NB, B), jnp.float32)],
    )(xt, tgt2d)
    return out[0, 0]
